# Initial kernel scaffold; baseline (speedup 1.0000x reference)
import jax
import jax.numpy as jnp
from jax.experimental import pallas as pl


def kernel(x, adj, W1, b1, W2, b2, W3, b3):
    raise NotImplementedError("write your pallas kernel here")


# speedup vs baseline: 1.2013x; 1.2013x over previous
import functools

import jax
import jax.numpy as jnp
from jax import lax
from jax.experimental import pallas as pl

_F32 = jnp.float32


def _mm_body(a_ref, b_ref, o_ref, *, act):
    o = jnp.dot(a_ref[...], b_ref[...], preferred_element_type=_F32)
    if act:
        o = jnp.maximum(o, 0.0)
    o_ref[...] = o


def _mm_bias_body(a_ref, b_ref, bias_ref, o_ref, *, act):
    o = jnp.dot(a_ref[...], b_ref[...], preferred_element_type=_F32) + bias_ref[...]
    if act:
        o = jnp.maximum(o, 0.0)
    o_ref[...] = o


def _mm(a, b, bias=None, act=False, block_m=512):
    m, k = a.shape
    _, n = b.shape
    in_specs = [
        pl.BlockSpec((block_m, k), lambda i: (i, 0)),
        pl.BlockSpec((k, n), lambda i: (0, 0)),
    ]
    args = [a, b]
    if bias is not None:
        in_specs.append(pl.BlockSpec((1, n), lambda i: (0, 0)))
        args.append(bias.reshape(1, n))
        body = functools.partial(_mm_bias_body, act=act)
    else:
        body = functools.partial(_mm_body, act=act)
    return pl.pallas_call(
        body,
        grid=(m // block_m,),
        in_specs=in_specs,
        out_specs=pl.BlockSpec((block_m, n), lambda i: (i, 0)),
        out_shape=jax.ShapeDtypeStruct((m, n), _F32),
    )(*args)


def _mm2_body(a_ref, b1_ref, b2_ref, c1_ref, c2_ref, o1_ref, o2_ref):
    a = a_ref[...]
    o1_ref[...] = (
        jnp.dot(a, b1_ref[...], preferred_element_type=_F32) + c1_ref[...]
    )
    o2_ref[...] = (
        jnp.dot(a, b2_ref[...], preferred_element_type=_F32) + c2_ref[...]
    )


def _mm2(a, b1, b2, c1, c2, block_m=256):
    m, k = a.shape
    _, n = b1.shape
    return pl.pallas_call(
        _mm2_body,
        grid=(m // block_m,),
        in_specs=[
            pl.BlockSpec((block_m, k), lambda i: (i, 0)),
            pl.BlockSpec((k, n), lambda i: (0, 0)),
            pl.BlockSpec((k, n), lambda i: (0, 0)),
            pl.BlockSpec((1, n), lambda i: (0, 0)),
            pl.BlockSpec((1, n), lambda i: (0, 0)),
        ],
        out_specs=[
            pl.BlockSpec((block_m, n), lambda i: (i, 0)),
            pl.BlockSpec((block_m, n), lambda i: (i, 0)),
        ],
        out_shape=[
            jax.ShapeDtypeStruct((m, n), _F32),
            jax.ShapeDtypeStruct((m, n), _F32),
        ],
    )(a, b1, b2, c1.reshape(1, n), c2.reshape(1, n))


def _lvz_body(a_ref, b_ref, mu_ref, eps_ref, lv_ref, z_ref):
    lv = jnp.dot(a_ref[...], b_ref[...], preferred_element_type=_F32)
    lv_ref[...] = lv
    z_ref[...] = mu_ref[...] + eps_ref[...] * jnp.exp(0.5 * lv)


def _lvz(g2, mu, eps, block_m=256):
    n = g2.shape[0]
    return pl.pallas_call(
        _lvz_body,
        grid=(n // block_m,),
        in_specs=[
            pl.BlockSpec((block_m, n), lambda i: (i, 0)),
            pl.BlockSpec((n, n), lambda i: (0, 0)),
            pl.BlockSpec((block_m, n), lambda i: (i, 0)),
            pl.BlockSpec((block_m, n), lambda i: (i, 0)),
        ],
        out_specs=[
            pl.BlockSpec((block_m, n), lambda i: (i, 0)),
            pl.BlockSpec((block_m, n), lambda i: (i, 0)),
        ],
        out_shape=[
            jax.ShapeDtypeStruct((n, n), _F32),
            jax.ShapeDtypeStruct((n, n), _F32),
        ],
    )(g2, g2, mu, eps)


def _mm_nt_body(a_ref, b_ref, o_ref):
    o_ref[...] = lax.dot_general(
        a_ref[...],
        b_ref[...],
        (((1,), (1,)), ((), ())),
        preferred_element_type=_F32,
    )


def _mm_nt(a, b, block_m=512):
    m, k = a.shape
    n = b.shape[0]
    return pl.pallas_call(
        _mm_nt_body,
        grid=(m // block_m,),
        in_specs=[
            pl.BlockSpec((block_m, k), lambda i: (i, 0)),
            pl.BlockSpec((n, k), lambda i: (0, 0)),
        ],
        out_specs=pl.BlockSpec((block_m, n), lambda i: (i, 0)),
        out_shape=jax.ShapeDtypeStruct((m, n), _F32),
    )(a, b)


def kernel(x, adj, W1, b1, W2, b2, W3, b3):
    n = adj.shape[0]
    xw = _mm(x, W3)
    h1 = _mm(adj, xw, bias=b3, act=True)
    t1, t2 = _mm2(h1, W1, W2, jnp.zeros_like(b1), jnp.zeros_like(b2))
    g1, g2 = _mm2(adj, t1, t2, b1, b2)
    mu = _mm(g1, g1)
    eps = jax.random.uniform(jax.random.key(42), (n, n), dtype=_F32)
    logvar, z = _lvz(g2, mu, eps)
    zz = _mm_nt(z, z)
    y = _mm(zz, zz, act=True)
    return (mu, logvar, y)


# speedup vs baseline: 1.2426x; 1.0344x over previous
import functools

import jax
import jax.numpy as jnp
from jax import lax
from jax.experimental import pallas as pl

_F32 = jnp.float32


def _mm_body(a_ref, b_ref, o_ref, *, act):
    o = jnp.dot(a_ref[...], b_ref[...], preferred_element_type=_F32)
    if act:
        o = jnp.maximum(o, 0.0)
    o_ref[...] = o


def _mm_bias_body(a_ref, b_ref, bias_ref, o_ref, *, act):
    o = jnp.dot(a_ref[...], b_ref[...], preferred_element_type=_F32) + bias_ref[...]
    if act:
        o = jnp.maximum(o, 0.0)
    o_ref[...] = o


def _mm(a, b, bias=None, act=False, block_m=512):
    m, k = a.shape
    _, n = b.shape
    in_specs = [
        pl.BlockSpec((block_m, k), lambda i: (i, 0)),
        pl.BlockSpec((k, n), lambda i: (0, 0)),
    ]
    args = [a, b]
    if bias is not None:
        in_specs.append(pl.BlockSpec((1, n), lambda i: (0, 0)))
        args.append(bias.reshape(1, n))
        body = functools.partial(_mm_bias_body, act=act)
    else:
        body = functools.partial(_mm_body, act=act)
    return pl.pallas_call(
        body,
        grid=(m // block_m,),
        in_specs=in_specs,
        out_specs=pl.BlockSpec((block_m, n), lambda i: (i, 0)),
        out_shape=jax.ShapeDtypeStruct((m, n), _F32),
    )(*args)


def _mm2_body(a_ref, b1_ref, b2_ref, c1_ref, c2_ref, o1_ref, o2_ref):
    a = a_ref[...]
    o1_ref[...] = (
        jnp.dot(a, b1_ref[...], preferred_element_type=_F32) + c1_ref[...]
    )
    o2_ref[...] = (
        jnp.dot(a, b2_ref[...], preferred_element_type=_F32) + c2_ref[...]
    )


def _mm2(a, b1, b2, c1, c2, block_m=256):
    m, k = a.shape
    _, n = b1.shape
    return pl.pallas_call(
        _mm2_body,
        grid=(m // block_m,),
        in_specs=[
            pl.BlockSpec((block_m, k), lambda i: (i, 0)),
            pl.BlockSpec((k, n), lambda i: (0, 0)),
            pl.BlockSpec((k, n), lambda i: (0, 0)),
            pl.BlockSpec((1, n), lambda i: (0, 0)),
            pl.BlockSpec((1, n), lambda i: (0, 0)),
        ],
        out_specs=[
            pl.BlockSpec((block_m, n), lambda i: (i, 0)),
            pl.BlockSpec((block_m, n), lambda i: (i, 0)),
        ],
        out_shape=[
            jax.ShapeDtypeStruct((m, n), _F32),
            jax.ShapeDtypeStruct((m, n), _F32),
        ],
    )(a, b1, b2, c1.reshape(1, n), c2.reshape(1, n))


def _lvz_body(a_ref, b_ref, mu_ref, eps_ref, lv_ref, z_ref):
    lv = jnp.dot(a_ref[...], b_ref[...], preferred_element_type=_F32)
    lv_ref[...] = lv
    z = mu_ref[...] + eps_ref[...] * jnp.exp(0.5 * lv)
    z_ref[...] = z.astype(jnp.bfloat16)


def _lvz(g2, mu, eps, block_m=256):
    n = g2.shape[0]
    return pl.pallas_call(
        _lvz_body,
        grid=(n // block_m,),
        in_specs=[
            pl.BlockSpec((block_m, n), lambda i: (i, 0)),
            pl.BlockSpec((n, n), lambda i: (0, 0)),
            pl.BlockSpec((block_m, n), lambda i: (i, 0)),
            pl.BlockSpec((block_m, n), lambda i: (i, 0)),
        ],
        out_specs=[
            pl.BlockSpec((block_m, n), lambda i: (i, 0)),
            pl.BlockSpec((block_m, n), lambda i: (i, 0)),
        ],
        out_shape=[
            jax.ShapeDtypeStruct((n, n), _F32),
            jax.ShapeDtypeStruct((n, n), jnp.bfloat16),
        ],
    )(g2, g2, mu, eps)


def _mm_nt_body(a_ref, b_ref, o_ref):
    o = lax.dot_general(
        a_ref[...],
        b_ref[...],
        (((1,), (1,)), ((), ())),
        preferred_element_type=_F32,
    )
    o_ref[...] = o.astype(o_ref.dtype)


def _mm_nt(a, b, out_dtype=_F32, block_m=512):
    m, k = a.shape
    n = b.shape[0]
    return pl.pallas_call(
        _mm_nt_body,
        grid=(m // block_m,),
        in_specs=[
            pl.BlockSpec((block_m, k), lambda i: (i, 0)),
            pl.BlockSpec((n, k), lambda i: (0, 0)),
        ],
        out_specs=pl.BlockSpec((block_m, n), lambda i: (i, 0)),
        out_shape=jax.ShapeDtypeStruct((m, n), out_dtype),
    )(a, b)


def kernel(x, adj, W1, b1, W2, b2, W3, b3):
    n = adj.shape[0]
    xw = _mm(x, W3)
    h1 = _mm(adj, xw, bias=b3, act=True)
    t1, t2 = _mm2(h1, W1, W2, jnp.zeros_like(b1), jnp.zeros_like(b2))
    g1, g2 = _mm2(adj, t1, t2, b1, b2)
    mu = _mm(g1, g1)
    eps = jax.random.uniform(jax.random.key(42), (n, n), dtype=_F32)
    logvar, z = _lvz(g2, mu, eps)
    zz = _mm_nt(z, z, out_dtype=jnp.bfloat16)
    y = _mm(zz, zz, act=True)
    return (mu, logvar, y)


# speedup vs baseline: 1.3877x; 1.1167x over previous
import functools

import jax
import jax.numpy as jnp
from jax import lax
from jax.experimental import pallas as pl

_F32 = jnp.float32
_BF16 = jnp.bfloat16


def _dot(a, b, trans_b=False):
    dims = (((1,), (1 if trans_b else 0,)), ((), ()))
    return lax.dot_general(
        a.astype(_BF16), b.astype(_BF16), dims, preferred_element_type=_F32
    )


def _mm_body(a_ref, b_ref, o_ref, *, act):
    o = _dot(a_ref[...], b_ref[...])
    if act:
        o = jnp.maximum(o, 0.0)
    o_ref[...] = o.astype(o_ref.dtype)


def _mm_bias_body(a_ref, b_ref, bias_ref, o_ref, *, act):
    o = _dot(a_ref[...], b_ref[...]) + bias_ref[...]
    if act:
        o = jnp.maximum(o, 0.0)
    o_ref[...] = o.astype(o_ref.dtype)


def _mm(a, b, bias=None, act=False, out_dtype=_F32, block_m=512):
    m, k = a.shape
    _, n = b.shape
    in_specs = [
        pl.BlockSpec((block_m, k), lambda i: (i, 0)),
        pl.BlockSpec((k, n), lambda i: (0, 0)),
    ]
    args = [a, b]
    if bias is not None:
        in_specs.append(pl.BlockSpec((1, n), lambda i: (0, 0)))
        args.append(bias.reshape(1, n))
        body = functools.partial(_mm_bias_body, act=act)
    else:
        body = functools.partial(_mm_body, act=act)
    return pl.pallas_call(
        body,
        grid=(m // block_m,),
        in_specs=in_specs,
        out_specs=pl.BlockSpec((block_m, n), lambda i: (i, 0)),
        out_shape=jax.ShapeDtypeStruct((m, n), out_dtype),
    )(*args)


def _mm2_body(a_ref, b1_ref, b2_ref, c1_ref, c2_ref, o1_ref, o2_ref):
    a = a_ref[...].astype(_BF16)
    o1 = _dot(a, b1_ref[...]) + c1_ref[...]
    o2 = _dot(a, b2_ref[...]) + c2_ref[...]
    o1_ref[...] = o1.astype(o1_ref.dtype)
    o2_ref[...] = o2.astype(o2_ref.dtype)


def _mm2(a, b1, b2, c1, c2, out_dtype=_F32, block_m=512):
    m, k = a.shape
    _, n = b1.shape
    return pl.pallas_call(
        _mm2_body,
        grid=(m // block_m,),
        in_specs=[
            pl.BlockSpec((block_m, k), lambda i: (i, 0)),
            pl.BlockSpec((k, n), lambda i: (0, 0)),
            pl.BlockSpec((k, n), lambda i: (0, 0)),
            pl.BlockSpec((1, n), lambda i: (0, 0)),
            pl.BlockSpec((1, n), lambda i: (0, 0)),
        ],
        out_specs=[
            pl.BlockSpec((block_m, n), lambda i: (i, 0)),
            pl.BlockSpec((block_m, n), lambda i: (i, 0)),
        ],
        out_shape=[
            jax.ShapeDtypeStruct((m, n), out_dtype),
            jax.ShapeDtypeStruct((m, n), out_dtype),
        ],
    )(a, b1, b2, c1.reshape(1, n), c2.reshape(1, n))


def _muz_body(g1a_ref, g1b_ref, g2a_ref, g2b_ref, eps_ref, mu_ref, lv_ref, z_ref):
    mu = _dot(g1a_ref[...], g1b_ref[...])
    lv = _dot(g2a_ref[...], g2b_ref[...])
    mu_ref[...] = mu
    lv_ref[...] = lv
    z = mu + eps_ref[...].astype(_F32) * jnp.exp(0.5 * lv)
    z_ref[...] = z.astype(_BF16)


def _muz(g1, g2, eps, block_m=256):
    n = g1.shape[0]
    row = pl.BlockSpec((block_m, n), lambda i: (i, 0))
    full = pl.BlockSpec((n, n), lambda i: (0, 0))
    return pl.pallas_call(
        _muz_body,
        grid=(n // block_m,),
        in_specs=[row, full, row, full, row],
        out_specs=[row, row, row],
        out_shape=[
            jax.ShapeDtypeStruct((n, n), _F32),
            jax.ShapeDtypeStruct((n, n), _F32),
            jax.ShapeDtypeStruct((n, n), _BF16),
        ],
    )(g1, g1, g2, g2, eps)


def _mm_nt_body(a_ref, b_ref, o_ref):
    o = _dot(a_ref[...], b_ref[...], trans_b=True)
    o_ref[...] = o.astype(o_ref.dtype)


def _mm_nt(a, b, out_dtype=_F32, block_m=512):
    m, k = a.shape
    n = b.shape[0]
    return pl.pallas_call(
        _mm_nt_body,
        grid=(m // block_m,),
        in_specs=[
            pl.BlockSpec((block_m, k), lambda i: (i, 0)),
            pl.BlockSpec((n, k), lambda i: (0, 0)),
        ],
        out_specs=pl.BlockSpec((block_m, n), lambda i: (i, 0)),
        out_shape=jax.ShapeDtypeStruct((m, n), out_dtype),
    )(a, b)


def kernel(x, adj, W1, b1, W2, b2, W3, b3):
    n = adj.shape[0]
    xw = _mm(x, W3, out_dtype=_BF16)
    h1 = _mm(adj, xw, bias=b3, act=True, out_dtype=_BF16)
    zero = jnp.zeros_like(b1)
    t1, t2 = _mm2(h1, W1, W2, zero, zero, out_dtype=_BF16)
    g1, g2 = _mm2(adj, t1, t2, b1, b2, out_dtype=_BF16)
    eps = jax.random.uniform(jax.random.key(42), (n, n), dtype=_F32)
    mu, logvar, z = _muz(g1, g2, eps.astype(_BF16))
    zz = _mm_nt(z, z, out_dtype=_BF16)
    y = _mm(zz, zz, act=True)
    return (mu, logvar, y)
